# Initial kernel scaffold; baseline (speedup 1.0000x reference)
import jax
import jax.numpy as jnp
from jax.experimental import pallas as pl


def kernel(input, mask):
    raise NotImplementedError("write your pallas kernel here")


# speedup vs baseline: 1.2695x; 1.2695x over previous
import jax
import jax.numpy as jnp
from jax.experimental import pallas as pl


def _shift_body(lat_ref, fmr_ref, frow_ref, fcol_ref, out_ref):
    lat = lat_ref[0]
    fmr = fmr_ref[0]
    frow = frow_ref[0]
    fcol = fcol_ref[0]
    n = lat.shape[0]

    norm = jnp.sqrt(jnp.sum(lat * lat, axis=1, keepdims=True)) + 1e-8
    latn = lat / norm
    sim = jax.lax.dot_general(
        latn, latn, (((1,), (1,)), ((), ())),
        preferred_element_type=jnp.float32,
        precision=jax.lax.Precision.DEFAULT)
    sim = jnp.where(frow > 0, jnp.float32(-1e9), sim)
    rowmax = jnp.max(sim, axis=1, keepdims=True)
    kiota = jax.lax.broadcasted_iota(jnp.int32, (n, n), 1)
    idx = jnp.min(jnp.where(sim == rowmax, kiota, n), axis=1,
                  keepdims=True)
    niota = jax.lax.broadcasted_iota(jnp.int32, (n, 1), 0)
    sel = jnp.where(fcol > 0, idx, niota)
    onehot = (sel == kiota).astype(jnp.float32)
    out_ref[0] = jax.lax.dot_general(
        onehot, fmr, (((1,), (0,)), ((), ())),
        preferred_element_type=jnp.float32,
        precision=jax.lax.Precision.HIGHEST)


def kernel(input, mask):
    b, c, h, w = input.shape
    ch = c // 2
    n = h * w
    x = input.reshape(b, c, n)
    fmr = x[:, :ch].transpose(0, 2, 1)
    lat = x[:, ch:].transpose(0, 2, 1)
    frow = (mask.reshape(1, 1, n) >= 1).astype(jnp.int32)
    fcol = frow.reshape(1, n, 1)

    shifted = pl.pallas_call(
        _shift_body,
        grid=(b,),
        in_specs=[
            pl.BlockSpec((1, n, ch), lambda i: (i, 0, 0)),
            pl.BlockSpec((1, n, ch), lambda i: (i, 0, 0)),
            pl.BlockSpec((1, 1, n), lambda i: (0, 0, 0)),
            pl.BlockSpec((1, n, 1), lambda i: (0, 0, 0)),
        ],
        out_specs=pl.BlockSpec((1, n, ch), lambda i: (i, 0, 0)),
        out_shape=jax.ShapeDtypeStruct((b, n, ch), jnp.float32),
    )(lat, fmr, frow, fcol)

    shifted = shifted.transpose(0, 2, 1).reshape(b, ch, h, w)
    return jnp.concatenate([input, shifted], axis=1)
